# baseline (speedup 1.0000x reference)
import functools

import jax
import jax.numpy as jnp
from jax import lax
from jax.experimental import pallas as pl
from jax.experimental.pallas import tpu as pltpu
from jax.experimental.pallas import tpu_sc as plsc

VOCAB = 100000
EMBED = 128
BATCH = 4096
HIST = 50

NC = 2
NS = 16
NW = NC * NS
NI = BATCH // NW
NBUF = 5
LA = 3

_mesh = plsc.VectorSubcoreMesh(core_axis_name="c", subcore_axis_name="s")


@functools.partial(
    pl.kernel,
    out_type=jax.ShapeDtypeStruct((HIST, BATCH, EMBED), jnp.float32),
    mesh=_mesh,
    scratch_types=[
        pltpu.VMEM((HIST, NI), jnp.int32),
        pltpu.VMEM((NBUF, NI, EMBED), jnp.float32),
        pltpu.SemaphoreType.DMA((NBUF,)),
        pltpu.SemaphoreType.DMA((NBUF,)),
    ],
)
def _sc_gather(idx_hbm, table_hbm, out_hbm, idx_v, rows_v, gsem, wsem):
    wid = lax.axis_index("s") * NC + lax.axis_index("c")
    base = wid * NI
    pltpu.sync_copy(idx_hbm.at[:, wid], idx_v)

    def gather(h):
        b = h % NBUF
        return pltpu.make_async_copy(
            table_hbm.at[idx_v.at[h]], rows_v.at[b], gsem.at[b])

    def write(h):
        b = h % NBUF
        return pltpu.make_async_copy(
            rows_v.at[b], out_hbm.at[h].at[pl.ds(base, NI)], wsem.at[b])

    NG = HIST // NBUF

    def step(h, jg, jf):
        pltpu.make_async_copy(
            rows_v.at[jf], out_hbm.at[h - NBUF + LA].at[pl.ds(base, NI)],
            wsem.at[jf]).wait()
        pltpu.make_async_copy(
            table_hbm.at[idx_v.at[h + LA]], rows_v.at[jf], gsem.at[jf]).start()
        gather_wait(h, jg)
        write_start(h, jg)

    def gather_wait(h, j):
        pltpu.make_async_copy(
            table_hbm.at[idx_v.at[h]], rows_v.at[j], gsem.at[j]).wait()

    def write_start(h, j):
        pltpu.make_async_copy(
            rows_v.at[j], out_hbm.at[h].at[pl.ds(base, NI)], wsem.at[j]).start()

    for h in range(LA):
        gather(h).start()
    for h in range(NBUF):
        f = h + LA
        if f >= NBUF:
            write(f - NBUF).wait()
        gather(f).start()
        gather(h).wait()
        write(h).start()

    def group(g, _):
        h0 = g * NBUF
        for j in range(NBUF):
            step(h0 + j, j, (j + LA) % NBUF)
        return ()

    lax.fori_loop(1, NG - 1, group, (), unroll=False)
    for h in range((NG - 1) * NBUF, HIST):
        f = h + LA
        if f < HIST:
            write(f - NBUF).wait()
            gather(f).start()
        gather(h).wait()
        write(h).start()
    for h in range(HIST - NBUF, HIST):
        write(h).wait()


def kernel(inputs, weight):
    idx3 = inputs.T.astype(jnp.int32).reshape(HIST, NW, NI)
    out = _sc_gather(idx3, weight)
    return out.transpose(1, 0, 2)
